# baseline (speedup 1.0000x reference)
import jax
import jax.numpy as jnp
from jax.experimental import pallas as pl
from jax.experimental.pallas import tpu as pltpu

B, V = 16384, 1000
H = B // 2


def _body(ia_ref, ib_ref, ig_ref, a_hbm, b_hbm, g_hbm, o_ref,
          a_v, b_v, g_v, sem0, sem1):
    lo, hi = [], []
    for hbm, idx_ref, v in ((a_hbm, ia_ref, a_v), (b_hbm, ib_ref, b_v),
                            (g_hbm, ig_ref, g_v)):
        row = hbm.at[pl.ds(idx_ref[0], 1)]
        cp0 = pltpu.make_async_copy(row.at[:, pl.ds(0, H)], v.at[:, pl.ds(0, H)], sem0)
        cp1 = pltpu.make_async_copy(row.at[:, pl.ds(H, H)], v.at[:, pl.ds(H, H)], sem1)
        cp0.start(priority=0)
        cp1.start(priority=1)
        lo.append(cp0)
        hi.append(cp1)
    for cp in lo:
        cp.wait()
    o_ref[pl.ds(0, H)] = jnp.maximum(
        a_v[0, pl.ds(0, H)] + b_v[0, pl.ds(0, H)] - g_v[0, pl.ds(0, H)], 0.0)
    for cp in hi:
        cp.wait()
    o_ref[pl.ds(H, H)] = jnp.maximum(
        a_v[0, pl.ds(H, H)] + b_v[0, pl.ds(H, H)] - g_v[0, pl.ds(H, H)], 0.0)


_call = pl.pallas_call(
    _body,
    in_specs=[
        pl.BlockSpec(memory_space=pltpu.MemorySpace.SMEM),
        pl.BlockSpec(memory_space=pltpu.MemorySpace.SMEM),
        pl.BlockSpec(memory_space=pltpu.MemorySpace.SMEM),
        pl.BlockSpec(memory_space=pltpu.MemorySpace.HBM),
        pl.BlockSpec(memory_space=pltpu.MemorySpace.HBM),
        pl.BlockSpec(memory_space=pltpu.MemorySpace.HBM),
    ],
    out_specs=pl.BlockSpec(memory_space=pltpu.MemorySpace.VMEM),
    out_shape=jax.ShapeDtypeStruct((B,), jnp.float32),
    scratch_shapes=[
        pltpu.VMEM((1, B), jnp.float32),
        pltpu.VMEM((1, B), jnp.float32),
        pltpu.VMEM((1, B), jnp.float32),
        pltpu.SemaphoreType.DMA,
        pltpu.SemaphoreType.DMA,
    ],
)


def kernel(log_y_alpha, log_y_beta, log_y_gamma, alpha_index, beta_index, gamma_index):
    ia = jnp.full((1,), alpha_index, dtype=jnp.int32)
    ib = jnp.full((1,), beta_index, dtype=jnp.int32)
    ig = jnp.full((1,), gamma_index, dtype=jnp.int32)
    return _call(ia, ib, ig, log_y_alpha.T, log_y_beta.T, log_y_gamma.T)


# speedup vs baseline: 1.0315x; 1.0315x over previous
import jax
import jax.numpy as jnp
from jax.experimental import pallas as pl
from jax.experimental.pallas import tpu as pltpu

B, V = 16384, 1000
H = B // 2


def _body(ia_ref, ib_ref, ig_ref, a_hbm, b_hbm, g_hbm, o_hbm,
          a_v, b_v, g_v, o_v, sem0, sem1, semo):
    lo, hi = [], []
    for hbm, idx_ref, v in ((a_hbm, ia_ref, a_v), (b_hbm, ib_ref, b_v),
                            (g_hbm, ig_ref, g_v)):
        row = hbm.at[pl.ds(idx_ref[0], 1)]
        cp0 = pltpu.make_async_copy(row.at[:, pl.ds(0, H)], v.at[:, pl.ds(0, H)], sem0)
        cp1 = pltpu.make_async_copy(row.at[:, pl.ds(H, H)], v.at[:, pl.ds(H, H)], sem1)
        cp0.start(priority=0)
        cp1.start(priority=1)
        lo.append(cp0)
        hi.append(cp1)
    for cp in lo:
        cp.wait()
    o_v[pl.ds(0, H)] = jnp.maximum(
        a_v[0, pl.ds(0, H)] + b_v[0, pl.ds(0, H)] - g_v[0, pl.ds(0, H)], 0.0)
    out0 = pltpu.make_async_copy(o_v.at[pl.ds(0, H)], o_hbm.at[pl.ds(0, H)], semo)
    out0.start(priority=0)
    for cp in hi:
        cp.wait()
    o_v[pl.ds(H, H)] = jnp.maximum(
        a_v[0, pl.ds(H, H)] + b_v[0, pl.ds(H, H)] - g_v[0, pl.ds(H, H)], 0.0)
    out1 = pltpu.make_async_copy(o_v.at[pl.ds(H, H)], o_hbm.at[pl.ds(H, H)], semo)
    out1.start(priority=1)
    out0.wait()
    out1.wait()


_call = pl.pallas_call(
    _body,
    in_specs=[
        pl.BlockSpec(memory_space=pltpu.MemorySpace.SMEM),
        pl.BlockSpec(memory_space=pltpu.MemorySpace.SMEM),
        pl.BlockSpec(memory_space=pltpu.MemorySpace.SMEM),
        pl.BlockSpec(memory_space=pltpu.MemorySpace.HBM),
        pl.BlockSpec(memory_space=pltpu.MemorySpace.HBM),
        pl.BlockSpec(memory_space=pltpu.MemorySpace.HBM),
    ],
    out_specs=pl.BlockSpec(memory_space=pltpu.MemorySpace.HBM),
    out_shape=jax.ShapeDtypeStruct((B,), jnp.float32),
    scratch_shapes=[
        pltpu.VMEM((1, B), jnp.float32),
        pltpu.VMEM((1, B), jnp.float32),
        pltpu.VMEM((1, B), jnp.float32),
        pltpu.VMEM((B,), jnp.float32),
        pltpu.SemaphoreType.DMA,
        pltpu.SemaphoreType.DMA,
        pltpu.SemaphoreType.DMA,
    ],
)


def kernel(log_y_alpha, log_y_beta, log_y_gamma, alpha_index, beta_index, gamma_index):
    ia = jnp.full((1,), alpha_index, dtype=jnp.int32)
    ib = jnp.full((1,), beta_index, dtype=jnp.int32)
    ig = jnp.full((1,), gamma_index, dtype=jnp.int32)
    return _call(ia, ib, ig, log_y_alpha.T, log_y_beta.T, log_y_gamma.T)
